# baseline (speedup 1.0000x reference)
import functools

import jax
import jax.numpy as jnp
from jax import lax
from jax.experimental import pallas as pl
from jax.experimental.pallas import tpu as pltpu
from jax.experimental.pallas import tpu_sc as plsc

B = 8
NL = 13
L = 512
D = 768
NBINS = 320
_LANES = 16
_NCHUNK = L // _LANES
_NZERO = NBINS // _LANES

_mesh = plsc.VectorSubcoreMesh(core_axis_name="c", subcore_axis_name="s")


@functools.partial(
    pl.kernel,
    mesh=_mesh,
    compiler_params=pltpu.CompilerParams(needs_layout_passes=False),
    out_type=jax.ShapeDtypeStruct((B, L), jnp.float32),
    scratch_types=[
        pltpu.VMEM((L,), jnp.int32),
        pltpu.VMEM((L,), jnp.int32),
        pltpu.VMEM((NBINS,), jnp.int32),
        pltpu.VMEM((NBINS,), jnp.int32),
        pltpu.VMEM((L,), jnp.float32),
    ],
)
def _sc_counts(ix_hbm, iy_hbm, f_hbm, ixv, iyv, hx, hy, fv):
    wid = lax.axis_index("s") * 2 + lax.axis_index("c")

    @pl.when(wid < B)
    def _():
        pltpu.sync_copy(ix_hbm.at[wid], ixv)
        pltpu.sync_copy(iy_hbm.at[wid], iyv)
        zeros = jnp.zeros((_LANES,), jnp.int32)
        for i in range(_NZERO):
            hx[pl.ds(i * _LANES, _LANES)] = zeros
            hy[pl.ds(i * _LANES, _LANES)] = zeros
        ones = jnp.full((_LANES,), 1, jnp.int32)
        for i in range(_NCHUNK):
            plsc.addupdate_scatter(hx, [ixv[pl.ds(i * _LANES, _LANES)]], ones)
            plsc.addupdate_scatter(hy, [iyv[pl.ds(i * _LANES, _LANES)]], ones)
        for i in range(_NCHUNK):
            fx = plsc.load_gather(hx, [ixv[pl.ds(i * _LANES, _LANES)]])
            fy = plsc.load_gather(hy, [iyv[pl.ds(i * _LANES, _LANES)]])
            fv[pl.ds(i * _LANES, _LANES)] = (fx + fy).astype(jnp.float32)
        pltpu.sync_copy(fv, f_hbm.at[wid])


def _pool_body(f_ref, x_ref, o_ref):
    r = 1.0 / f_ref[0]
    w = r * (1.0 / jnp.sum(r))
    x = x_ref[0, 0]
    o_ref[0] = jnp.sum(x * w[0, :, None], axis=0, keepdims=True)


def kernel(input_feature, input_lengths, vq_indices):
    del input_lengths
    vq = vq_indices.astype(jnp.int32)
    ix = vq[:, :, 0]
    iy = vq[:, :, 1]
    f = _sc_counts(ix, iy).reshape(B, 1, L)
    out = pl.pallas_call(
        _pool_body,
        grid=(B,),
        in_specs=[
            pl.BlockSpec((1, 1, L), lambda b: (b, 0, 0)),
            pl.BlockSpec((1, 1, L, D), lambda b: (b, NL - 1, 0, 0)),
        ],
        out_specs=pl.BlockSpec((1, 1, D), lambda b: (b, 0, 0)),
        out_shape=jax.ShapeDtypeStruct((B, 1, D), jnp.float32),
    )(f, input_feature)
    return out.reshape(B, D)


# speedup vs baseline: 2.1803x; 2.1803x over previous
import jax
import jax.numpy as jnp
from jax import lax
from jax.experimental import pallas as pl

B = 8
NL = 13
L = 512
D = 768
NBINS = 320

_HI = lax.Precision.HIGHEST


def _body(vq_ref, x_ref, o_ref):
    v = vq_ref[0]
    ixc = v[:, 0:1]
    iyc = v[:, 1:2]
    iota = lax.broadcasted_iota(jnp.int32, (L, NBINS), 1)
    ex = (ixc == iota).astype(jnp.float32)
    ey = (iyc == iota).astype(jnp.float32)
    cx = jnp.sum(ex, axis=0, keepdims=True)
    cy = jnp.sum(ey, axis=0, keepdims=True)
    fx = lax.dot_general(ex, cx, (((1,), (1,)), ((), ())), precision=_HI)
    fy = lax.dot_general(ey, cy, (((1,), (1,)), ((), ())), precision=_HI)
    r = 1.0 / (fx + fy)
    w = r * (1.0 / jnp.sum(r))
    x = x_ref[0, 0]
    o_ref[0] = jnp.sum(x * w, axis=0, keepdims=True)


def kernel(input_feature, input_lengths, vq_indices):
    del input_lengths
    vq = vq_indices.astype(jnp.int32)
    out = pl.pallas_call(
        _body,
        grid=(B,),
        in_specs=[
            pl.BlockSpec((1, L, 2), lambda b: (b, 0, 0)),
            pl.BlockSpec((1, 1, L, D), lambda b: (b, NL - 1, 0, 0)),
        ],
        out_specs=pl.BlockSpec((1, 1, D), lambda b: (b, 0, 0)),
        out_shape=jax.ShapeDtypeStruct((B, 1, D), jnp.float32),
    )(vq, input_feature)
    return out.reshape(B, D)
